# baseline (speedup 1.0000x reference)
import functools
import math

import jax
import jax.numpy as jnp
from jax import lax
from jax.experimental import pallas as pl
from jax.experimental.pallas import tpu as pltpu
from jax.experimental.pallas import tpu_sc as plsc

_SQRT2 = math.sqrt(2.0)

_L = 2048
_D = 1024
_H = 2048
_E = 8
_K = 2
_S = 2
_TG = 128
_NT = _L * _K // _TG + _E
_NP = _NT * _TG
_NW = 32
_TOK_W = _L // _NW
_ROW_W = _NP // _NW
_GB = 32


def _gelu(h):
    return 0.5 * h * (1.0 + lax.erf(h / _SQRT2))


def _router_body(x_ref, wr_ref, br_ref, te_ref, nact_ref, dest_ref, prob_ref):
    gates = x_ref[...] @ wr_ref[...] + br_ref[...]
    lanes = lax.broadcasted_iota(jnp.int32, gates.shape, 1)
    neg = jnp.float32(-1e30)
    big = jnp.int32(2**30)
    m1 = jnp.max(gates, axis=1, keepdims=True)
    i1 = jnp.min(jnp.where(gates == m1, lanes, big), axis=1, keepdims=True)
    g2 = jnp.where(lanes == i1, neg, gates)
    m2 = jnp.max(g2, axis=1, keepdims=True)
    i2 = jnp.min(jnp.where(g2 == m2, lanes, big), axis=1, keepdims=True)
    p1 = 1.0 / (1.0 + jnp.exp(m2 - m1))
    p2 = 1.0 - p1

    oh1 = (lanes == i1).astype(jnp.float32)
    oh2 = (lanes == i2).astype(jnp.float32)
    cnt = oh1 + oh2
    cinc = cnt
    s = 1
    while s < _L:
        cinc = cinc + jnp.concatenate(
            [jnp.zeros((s, _E), jnp.float32), cinc[:-s]], axis=0)
        s *= 2
    cexc = cinc - cnt
    counts = cinc[_L - 1:_L]
    ntiles = jnp.floor((counts + (_TG - 1)) * (1.0 / _TG))
    tri = (lax.broadcasted_iota(jnp.int32, (_E, _E), 0)
           <= lax.broadcasted_iota(jnp.int32, (_E, _E), 1)).astype(jnp.float32)
    bo_inc = ntiles @ tri
    bo_exc = bo_inc - ntiles
    off = bo_exc * float(_TG)

    rank1 = jnp.sum(oh1 * (cexc + off), axis=1, keepdims=True)
    rank2 = jnp.sum(oh2 * (cexc + off), axis=1, keepdims=True)
    dest_ref[...] = jnp.concatenate([rank1, rank2], axis=1).astype(jnp.int32)
    prob_ref[...] = jnp.concatenate([p1, p2], axis=1)

    eye = (lax.broadcasted_iota(jnp.int32, (_E, _E), 0)
           == lax.broadcasted_iota(jnp.int32, (_E, _E), 1)).astype(jnp.float32)
    bo_col = jnp.sum(jnp.broadcast_to(bo_inc, (_E, _E)) * eye,
                     axis=1, keepdims=True)
    ti = lax.broadcasted_iota(jnp.int32, (_E, _NT), 1).astype(jnp.float32)
    ge = (ti >= bo_col).astype(jnp.float32)
    te = jnp.ones((1, _E), jnp.float32) @ ge
    active = (ntiles > 0.0).astype(jnp.float32)
    max_e = jnp.max(
        lax.broadcasted_iota(jnp.int32, (1, _E), 1).astype(jnp.float32)
        * active, axis=1, keepdims=True)
    te_ref[...] = jnp.minimum(te, max_e).astype(jnp.int32)
    nact_ref[...] = jnp.sum(bo_inc * eye[_E - 1:_E], axis=1,
                            keepdims=True).astype(jnp.int32)


def _router(x2, Wr, br2):
    return pl.pallas_call(
        _router_body,
        in_specs=[
            pl.BlockSpec((_L, _D), lambda: (0, 0)),
            pl.BlockSpec((_D, _E), lambda: (0, 0)),
            pl.BlockSpec((1, _E), lambda: (0, 0)),
        ],
        out_specs=[
            pl.BlockSpec((1, _NT), lambda: (0, 0)),
            pl.BlockSpec((1, 1), lambda: (0, 0)),
            pl.BlockSpec((_L, _K), lambda: (0, 0)),
            pl.BlockSpec((_L, _K), lambda: (0, 0)),
        ],
        out_shape=[
            jax.ShapeDtypeStruct((1, _NT), jnp.int32),
            jax.ShapeDtypeStruct((1, 1), jnp.int32),
            jax.ShapeDtypeStruct((_L, _K), jnp.int32),
            jax.ShapeDtypeStruct((_L, _K), jnp.float32),
        ],
    )(x2, Wr, br2)


def _dispatch_body(x_hbm, dest_hbm, prob_hbm, xs_hbm, ps_hbm,
                   dest_v, prob_v, src_v, ps_v, rows_v, sem):
    wid = lax.axis_index("s") * 2 + lax.axis_index("c")
    base = wid * _ROW_W
    pltpu.sync_copy(dest_hbm, dest_v)
    pltpu.sync_copy(prob_hbm, prob_v)

    zi = jnp.zeros((16,), jnp.int32)
    zf = jnp.zeros((16,), jnp.float32)

    def zbody(c, _):
        src_v[pl.ds(c * 16, 16)] = zi
        ps_v[pl.ds(c * 16, 16)] = zf
        return 0
    lax.fori_loop(0, _NP // 16, zbody, 0)

    def sbody(c, _):
        off = c * 16
        idxv = dest_v[pl.ds(off, 16)]
        tok = lax.shift_right_logical(lax.iota(jnp.int32, 16) + off, 1)
        plsc.store_scatter(src_v, [idxv], tok)
        plsc.store_scatter(ps_v, [idxv], prob_v[pl.ds(off, 16)])
        return 0
    lax.fori_loop(0, _L * _K // 16, sbody, 0)

    pltpu.sync_copy(ps_v.at[pl.ds(base, _ROW_W)],
                    ps_hbm.at[pl.ds(base, _ROW_W)])

    for b in range(_ROW_W // _GB):
        r0 = base + b * _GB
        pltpu.async_copy(x_hbm.at[src_v.at[pl.ds(r0, _GB)]], rows_v, sem
                         ).wait()
        pltpu.sync_copy(rows_v, xs_hbm.at[pl.ds(r0, _GB)])


def _dispatch(x2, dest4, prob4):
    mesh = plsc.VectorSubcoreMesh(core_axis_name="c", subcore_axis_name="s")
    f = pl.kernel(
        _dispatch_body,
        out_type=[
            jax.ShapeDtypeStruct((_NP, _D), jnp.float32),
            jax.ShapeDtypeStruct((_NP,), jnp.float32),
        ],
        mesh=mesh,
        scratch_types=[
            pltpu.VMEM((_L * _K,), jnp.int32),
            pltpu.VMEM((_L * _K,), jnp.float32),
            pltpu.VMEM((_NP,), jnp.int32),
            pltpu.VMEM((_NP,), jnp.float32),
            pltpu.VMEM((_GB, _D), jnp.float32),
            pltpu.SemaphoreType.DMA,
        ],
        compiler_params=pltpu.CompilerParams(needs_layout_passes=False),
    )
    return f(x2, dest4, prob4)


def _shared_body(x_ref, w1_ref, b1_ref, w2_ref, b2_ref, out_ref, *, n_shared):
    p = pl.program_id(0)
    j = pl.program_id(1)

    @pl.when((p == 0) & (j == 0))
    def _():
        out_ref[...] = jnp.zeros_like(out_ref)

    h = x_ref[...] @ w1_ref[0] + b1_ref[0, 0]
    h = _gelu(h)
    y = h @ w2_ref[0]
    inv = 1.0 / n_shared

    @pl.when(j == 0)
    def _():
        out_ref[...] += b2_ref[0, 0] * inv

    out_ref[...] += y * inv


def _shared(x2, Ws1, bs1r, Ws2, bs2r):
    hc = 512
    jn = _H // hc
    return pl.pallas_call(
        functools.partial(_shared_body, n_shared=_S),
        grid=(_S, jn),
        in_specs=[
            pl.BlockSpec((_L, _D), lambda p, j: (0, 0)),
            pl.BlockSpec((1, _D, hc), lambda p, j: (p, 0, j)),
            pl.BlockSpec((1, 1, hc), lambda p, j: (p, 0, j)),
            pl.BlockSpec((1, hc, _D), lambda p, j: (p, j, 0)),
            pl.BlockSpec((1, 1, _D), lambda p, j: (p, 0, 0)),
        ],
        out_specs=pl.BlockSpec((_L, _D), lambda p, j: (0, 0)),
        out_shape=jax.ShapeDtypeStruct((_L, _D), jnp.float32),
    )(x2, Ws1, bs1r, Ws2, bs2r)


def _grouped_body(te_ref, nact_ref, xs_ref, ps_ref, w1_ref, b1_ref,
                  w2_ref, b2_ref, y_ref):
    i = pl.program_id(0)

    @pl.when(i < nact_ref[0])
    def _():
        h = xs_ref[...] @ w1_ref[0] + b1_ref[0, 0]
        h = _gelu(h)
        y = h @ w2_ref[0] + b2_ref[0, 0]
        y_ref[...] = y * ps_ref[...]


def _grouped(te, nact, Xs, Ps2, W1, b1r, W2, b2r):
    grid_spec = pltpu.PrefetchScalarGridSpec(
        num_scalar_prefetch=2,
        grid=(_NT,),
        in_specs=[
            pl.BlockSpec((_TG, _D), lambda i, te, na: (i, 0)),
            pl.BlockSpec((_TG, 1), lambda i, te, na: (i, 0)),
            pl.BlockSpec((1, _D, _H), lambda i, te, na: (te[i], 0, 0)),
            pl.BlockSpec((1, 1, _H), lambda i, te, na: (te[i], 0, 0)),
            pl.BlockSpec((1, _H, _D), lambda i, te, na: (te[i], 0, 0)),
            pl.BlockSpec((1, 1, _D), lambda i, te, na: (te[i], 0, 0)),
        ],
        out_specs=pl.BlockSpec((_TG, _D), lambda i, te, na: (i, 0)),
    )
    return pl.pallas_call(
        _grouped_body,
        grid_spec=grid_spec,
        out_shape=jax.ShapeDtypeStruct((_NP, _D), jnp.float32),
    )(te, nact, Xs, Ps2, W1, b1r, W2, b2r)


def _combine_body(y_hbm, dest_hbm, sh_hbm, out_hbm,
                  didx_v, prow_v, sh_v, out_v, sem):
    wid = lax.axis_index("s") * 2 + lax.axis_index("c")
    tok0 = wid * _TOK_W
    pltpu.sync_copy(dest_hbm.at[pl.ds(tok0 * _K, _TOK_W * _K)], didx_v)

    nb = _GB // _K
    for b in range(_TOK_W // nb):
        t0 = tok0 + b * nb
        pltpu.async_copy(y_hbm.at[didx_v.at[pl.ds(b * _GB, _GB)]],
                         prow_v, sem).wait()
        pltpu.sync_copy(sh_hbm.at[pl.ds(t0, nb)], sh_v)

        def jbody(j, _):
            def cbody(c, _):
                sl = pl.ds(c * 16, 16)
                out_v[j, sl] = (prow_v[2 * j, sl] + prow_v[2 * j + 1, sl]
                                + sh_v[j, sl])
                return 0
            lax.fori_loop(0, _D // 16, cbody, 0)
            return 0
        lax.fori_loop(0, nb, jbody, 0)

        pltpu.sync_copy(out_v, out_hbm.at[pl.ds(t0, nb)])


def _combine(Y, dest4, sh):
    mesh = plsc.VectorSubcoreMesh(core_axis_name="c", subcore_axis_name="s")
    nb = _GB // _K
    f = pl.kernel(
        _combine_body,
        out_type=jax.ShapeDtypeStruct((_L, _D), jnp.float32),
        mesh=mesh,
        scratch_types=[
            pltpu.VMEM((_TOK_W * _K,), jnp.int32),
            pltpu.VMEM((_GB, _D), jnp.float32),
            pltpu.VMEM((nb, _D), jnp.float32),
            pltpu.VMEM((nb, _D), jnp.float32),
            pltpu.SemaphoreType.DMA,
        ],
        compiler_params=pltpu.CompilerParams(needs_layout_passes=False),
    )
    return f(Y, dest4, sh)


def kernel(x, Wr, br, W1, b1, W2, b2, Ws1, bs1, Ws2, bs2):
    Bb, Ll, Dd = x.shape
    assert (Bb, Ll, Dd) == (1, _L, _D) and W1.shape == (_E, _D, _H)
    x2 = x.reshape(_L, _D)
    br2 = br.reshape(1, _E)
    b1r = b1.reshape(_E, 1, _H)
    b2r = b2.reshape(_E, 1, _D)
    bs1r = bs1.reshape(_S, 1, _H)
    bs2r = bs2.reshape(_S, 1, _D)

    te, nact, dest, prob = _router(x2, Wr, br2)
    dest4 = dest.reshape(_L * _K)
    prob4 = prob.reshape(_L * _K)
    Xs, Ps = _dispatch(x2, dest4, prob4)
    sh = _shared(x2, Ws1, bs1r, Ws2, bs2r)
    Y = _grouped(te.reshape(_NT), nact.reshape(1), Xs,
                 Ps.reshape(_NP, 1), W1, b1r, W2, b2r)
    out = _combine(Y, dest4, sh)
    return out.reshape(Bb, _L, _D)


# speedup vs baseline: 1.0367x; 1.0367x over previous
import functools
import math

import jax
import jax.numpy as jnp
from jax import lax
from jax.experimental import pallas as pl
from jax.experimental.pallas import tpu as pltpu
from jax.experimental.pallas import tpu_sc as plsc

_SQRT2 = math.sqrt(2.0)

_L = 2048
_D = 1024
_H = 2048
_E = 8
_K = 2
_S = 2
_TG = 128
_NT = _L * _K // _TG + _E
_NP = _NT * _TG
_NW = 32
_TOK_W = _L // _NW
_ROW_W = _NP // _NW
_GB = 32


def _gelu(h):
    return 0.5 * h * (1.0 + lax.erf(h / _SQRT2))


def _router_body(x_ref, wr_ref, br_ref, te_ref, nact_ref, dest_ref, prob_ref):
    gates = x_ref[...] @ wr_ref[...] + br_ref[...]
    lanes = lax.broadcasted_iota(jnp.int32, gates.shape, 1)
    neg = jnp.float32(-1e30)
    big = jnp.int32(2**30)
    m1 = jnp.max(gates, axis=1, keepdims=True)
    i1 = jnp.min(jnp.where(gates == m1, lanes, big), axis=1, keepdims=True)
    g2 = jnp.where(lanes == i1, neg, gates)
    m2 = jnp.max(g2, axis=1, keepdims=True)
    i2 = jnp.min(jnp.where(g2 == m2, lanes, big), axis=1, keepdims=True)
    p1 = 1.0 / (1.0 + jnp.exp(m2 - m1))
    p2 = 1.0 - p1

    oh1 = (lanes == i1).astype(jnp.float32)
    oh2 = (lanes == i2).astype(jnp.float32)
    cnt = oh1 + oh2
    cinc = cnt
    s = 1
    while s < _L:
        cinc = cinc + jnp.concatenate(
            [jnp.zeros((s, _E), jnp.float32), cinc[:-s]], axis=0)
        s *= 2
    cexc = cinc - cnt
    counts = cinc[_L - 1:_L]
    ntiles = jnp.floor((counts + (_TG - 1)) * (1.0 / _TG))
    tri = (lax.broadcasted_iota(jnp.int32, (_E, _E), 0)
           <= lax.broadcasted_iota(jnp.int32, (_E, _E), 1)).astype(jnp.float32)
    bo_inc = ntiles @ tri
    bo_exc = bo_inc - ntiles
    off = bo_exc * float(_TG)

    rank1 = jnp.sum(oh1 * (cexc + off), axis=1, keepdims=True)
    rank2 = jnp.sum(oh2 * (cexc + off), axis=1, keepdims=True)
    dest_ref[...] = jnp.concatenate([rank1, rank2], axis=1).astype(jnp.int32)
    prob_ref[...] = jnp.concatenate([p1, p2], axis=1)

    eye = (lax.broadcasted_iota(jnp.int32, (_E, _E), 0)
           == lax.broadcasted_iota(jnp.int32, (_E, _E), 1)).astype(jnp.float32)
    bo_col = jnp.sum(jnp.broadcast_to(bo_inc, (_E, _E)) * eye,
                     axis=1, keepdims=True)
    ti = lax.broadcasted_iota(jnp.int32, (_E, _NT), 1).astype(jnp.float32)
    ge = (ti >= bo_col).astype(jnp.float32)
    te = jnp.ones((1, _E), jnp.float32) @ ge
    active = (ntiles > 0.0).astype(jnp.float32)
    max_e = jnp.max(
        lax.broadcasted_iota(jnp.int32, (1, _E), 1).astype(jnp.float32)
        * active, axis=1, keepdims=True)
    te_ref[...] = jnp.minimum(te, max_e).astype(jnp.int32)
    nact_ref[...] = jnp.sum(bo_inc * eye[_E - 1:_E], axis=1,
                            keepdims=True).astype(jnp.int32)


def _router(x2, Wr, br2):
    return pl.pallas_call(
        _router_body,
        in_specs=[
            pl.BlockSpec((_L, _D), lambda: (0, 0)),
            pl.BlockSpec((_D, _E), lambda: (0, 0)),
            pl.BlockSpec((1, _E), lambda: (0, 0)),
        ],
        out_specs=[
            pl.BlockSpec((1, _NT), lambda: (0, 0)),
            pl.BlockSpec((1, 1), lambda: (0, 0)),
            pl.BlockSpec((_L, _K), lambda: (0, 0)),
            pl.BlockSpec((_L, _K), lambda: (0, 0)),
        ],
        out_shape=[
            jax.ShapeDtypeStruct((1, _NT), jnp.int32),
            jax.ShapeDtypeStruct((1, 1), jnp.int32),
            jax.ShapeDtypeStruct((_L, _K), jnp.int32),
            jax.ShapeDtypeStruct((_L, _K), jnp.float32),
        ],
    )(x2, Wr, br2)


def _dispatch_body(x_hbm, dest_hbm, prob_hbm, xs_hbm, ps_hbm,
                   dest_v, prob_v, src_v, ps_v, rows_v,
                   g0, g1, g2, w0, w1, w2, psem):
    wid = lax.axis_index("s") * 2 + lax.axis_index("c")
    base = wid * _ROW_W
    pltpu.sync_copy(dest_hbm, dest_v)
    pltpu.sync_copy(prob_hbm, prob_v)

    zi = jnp.zeros((16,), jnp.int32)
    zf = jnp.zeros((16,), jnp.float32)

    def zbody(c, _):
        src_v[pl.ds(c * 16, 16)] = zi
        ps_v[pl.ds(c * 16, 16)] = zf
        return 0
    lax.fori_loop(0, _NP // 16, zbody, 0)

    def sbody(c, _):
        off = c * 16
        idxv = dest_v[pl.ds(off, 16)]
        tok = lax.shift_right_logical(lax.iota(jnp.int32, 16) + off, 1)
        plsc.store_scatter(src_v, [idxv], tok)
        plsc.store_scatter(ps_v, [idxv], prob_v[pl.ds(off, 16)])
        return 0
    lax.fori_loop(0, _L * _K // 16, sbody, 0)

    ph = pltpu.async_copy(ps_v.at[pl.ds(base, _ROW_W)],
                          ps_hbm.at[pl.ds(base, _ROW_W)], psem)

    gsems = [g0, g1, g2]
    wsems = [w0, w1, w2]
    nbat = _ROW_W // _GB

    def gstart(b):
        return pltpu.async_copy(
            x_hbm.at[src_v.at[pl.ds(base + b * _GB, _GB)]],
            rows_v.at[b % 3], gsems[b % 3])

    gh = [None] * nbat
    wh = [None] * nbat
    for b in range(min(3, nbat)):
        gh[b] = gstart(b)
    for b in range(nbat):
        gh[b].wait()
        wh[b] = pltpu.async_copy(rows_v.at[b % 3],
                                 xs_hbm.at[pl.ds(base + b * _GB, _GB)],
                                 wsems[b % 3])
        if b + 3 < nbat:
            wh[b].wait()
            gh[b + 3] = gstart(b + 3)
    for b in range(max(0, nbat - 3), nbat):
        wh[b].wait()
    ph.wait()


def _dispatch(x2, dest4, prob4):
    mesh = plsc.VectorSubcoreMesh(core_axis_name="c", subcore_axis_name="s")
    f = pl.kernel(
        _dispatch_body,
        out_type=[
            jax.ShapeDtypeStruct((_NP, _D), jnp.float32),
            jax.ShapeDtypeStruct((_NP,), jnp.float32),
        ],
        mesh=mesh,
        scratch_types=[
            pltpu.VMEM((_L * _K,), jnp.int32),
            pltpu.VMEM((_L * _K,), jnp.float32),
            pltpu.VMEM((_NP,), jnp.int32),
            pltpu.VMEM((_NP,), jnp.float32),
            pltpu.VMEM((3, _GB, _D), jnp.float32),
            pltpu.SemaphoreType.DMA,
            pltpu.SemaphoreType.DMA,
            pltpu.SemaphoreType.DMA,
            pltpu.SemaphoreType.DMA,
            pltpu.SemaphoreType.DMA,
            pltpu.SemaphoreType.DMA,
            pltpu.SemaphoreType.DMA,
        ],
        compiler_params=pltpu.CompilerParams(needs_layout_passes=False),
    )
    return f(x2, dest4, prob4)


def _shared_body(x_ref, w1_ref, b1_ref, w2_ref, b2_ref, out_ref, *, n_shared):
    p = pl.program_id(0)
    j = pl.program_id(1)

    @pl.when((p == 0) & (j == 0))
    def _():
        out_ref[...] = jnp.zeros_like(out_ref)

    h = x_ref[...] @ w1_ref[0] + b1_ref[0, 0]
    h = _gelu(h)
    y = h @ w2_ref[0]
    inv = 1.0 / n_shared

    @pl.when(j == 0)
    def _():
        out_ref[...] += b2_ref[0, 0] * inv

    out_ref[...] += y * inv


def _shared(x2, Ws1, bs1r, Ws2, bs2r):
    hc = 512
    jn = _H // hc
    return pl.pallas_call(
        functools.partial(_shared_body, n_shared=_S),
        grid=(_S, jn),
        in_specs=[
            pl.BlockSpec((_L, _D), lambda p, j: (0, 0)),
            pl.BlockSpec((1, _D, hc), lambda p, j: (p, 0, j)),
            pl.BlockSpec((1, 1, hc), lambda p, j: (p, 0, j)),
            pl.BlockSpec((1, hc, _D), lambda p, j: (p, j, 0)),
            pl.BlockSpec((1, 1, _D), lambda p, j: (p, 0, 0)),
        ],
        out_specs=pl.BlockSpec((_L, _D), lambda p, j: (0, 0)),
        out_shape=jax.ShapeDtypeStruct((_L, _D), jnp.float32),
    )(x2, Ws1, bs1r, Ws2, bs2r)


def _grouped_body(te_ref, nact_ref, xs_ref, ps_ref, w1_ref, b1_ref,
                  w2_ref, b2_ref, y_ref):
    i = pl.program_id(0)

    @pl.when(i < nact_ref[0])
    def _():
        h = xs_ref[...] @ w1_ref[0] + b1_ref[0, 0]
        h = _gelu(h)
        y = h @ w2_ref[0] + b2_ref[0, 0]
        y_ref[...] = y * ps_ref[...]


def _grouped(te, nact, Xs, Ps2, W1, b1r, W2, b2r):
    grid_spec = pltpu.PrefetchScalarGridSpec(
        num_scalar_prefetch=2,
        grid=(_NT,),
        in_specs=[
            pl.BlockSpec((_TG, _D), lambda i, te, na: (i, 0)),
            pl.BlockSpec((_TG, 1), lambda i, te, na: (i, 0)),
            pl.BlockSpec((1, _D, _H), lambda i, te, na: (te[i], 0, 0)),
            pl.BlockSpec((1, 1, _H), lambda i, te, na: (te[i], 0, 0)),
            pl.BlockSpec((1, _H, _D), lambda i, te, na: (te[i], 0, 0)),
            pl.BlockSpec((1, 1, _D), lambda i, te, na: (te[i], 0, 0)),
        ],
        out_specs=pl.BlockSpec((_TG, _D), lambda i, te, na: (i, 0)),
    )
    return pl.pallas_call(
        _grouped_body,
        grid_spec=grid_spec,
        out_shape=jax.ShapeDtypeStruct((_NP, _D), jnp.float32),
    )(te, nact, Xs, Ps2, W1, b1r, W2, b2r)


_CNB = 8


def _combine_body(y_hbm, dest_hbm, sh_hbm, out_hbm,
                  didx_v, prow_v, sh_v, out_v,
                  ga0, ga1, gb0, gb1, wa0, wa1):
    wid = lax.axis_index("s") * 2 + lax.axis_index("c")
    tok0 = wid * _TOK_W
    pltpu.sync_copy(dest_hbm.at[pl.ds(tok0 * _K, _TOK_W * _K)], didx_v)

    gpsems = [ga0, ga1]
    gssems = [gb0, gb1]
    wsems = [wa0, wa1]
    nbat = _TOK_W // _CNB

    def istart(b):
        t0 = tok0 + b * _CNB
        hp = pltpu.async_copy(
            y_hbm.at[didx_v.at[pl.ds(b * _CNB * _K, _CNB * _K)]],
            prow_v.at[b % 2], gpsems[b % 2])
        hs = pltpu.async_copy(sh_hbm.at[pl.ds(t0, _CNB)], sh_v.at[b % 2],
                              gssems[b % 2])
        return hp, hs

    hp_ = [None] * nbat
    hs_ = [None] * nbat
    wh = [None] * nbat
    hp_[0], hs_[0] = istart(0)
    for b in range(nbat):
        if b + 1 < nbat:
            hp_[b + 1], hs_[b + 1] = istart(b + 1)
        hp_[b].wait()
        hs_[b].wait()
        if b >= 2:
            wh[b - 2].wait()
        bb = b % 2

        def jbody(j, _):
            for c in range(_D // 16):
                sl = pl.ds(c * 16, 16)
                out_v[bb, j, sl] = (prow_v[bb, 2 * j, sl]
                                    + prow_v[bb, 2 * j + 1, sl]
                                    + sh_v[bb, j, sl])
            return 0
        lax.fori_loop(0, _CNB, jbody, 0)

        wh[b] = pltpu.async_copy(out_v.at[bb],
                                 out_hbm.at[pl.ds(tok0 + b * _CNB, _CNB)],
                                 wsems[bb])
    wh[nbat - 2].wait()
    wh[nbat - 1].wait()


def _combine(Y, dest4, sh):
    mesh = plsc.VectorSubcoreMesh(core_axis_name="c", subcore_axis_name="s")
    f = pl.kernel(
        _combine_body,
        out_type=jax.ShapeDtypeStruct((_L, _D), jnp.float32),
        mesh=mesh,
        scratch_types=[
            pltpu.VMEM((_TOK_W * _K,), jnp.int32),
            pltpu.VMEM((2, _CNB * _K, _D), jnp.float32),
            pltpu.VMEM((2, _CNB, _D), jnp.float32),
            pltpu.VMEM((2, _CNB, _D), jnp.float32),
            pltpu.SemaphoreType.DMA,
            pltpu.SemaphoreType.DMA,
            pltpu.SemaphoreType.DMA,
            pltpu.SemaphoreType.DMA,
            pltpu.SemaphoreType.DMA,
            pltpu.SemaphoreType.DMA,
        ],
        compiler_params=pltpu.CompilerParams(needs_layout_passes=False),
    )
    return f(Y, dest4, sh)


def kernel(x, Wr, br, W1, b1, W2, b2, Ws1, bs1, Ws2, bs2):
    Bb, Ll, Dd = x.shape
    assert (Bb, Ll, Dd) == (1, _L, _D) and W1.shape == (_E, _D, _H)
    x2 = x.reshape(_L, _D)
    br2 = br.reshape(1, _E)
    b1r = b1.reshape(_E, 1, _H)
    b2r = b2.reshape(_E, 1, _D)
    bs1r = bs1.reshape(_S, 1, _H)
    bs2r = bs2.reshape(_S, 1, _D)

    te, nact, dest, prob = _router(x2, Wr, br2)
    dest4 = dest.reshape(_L * _K)
    prob4 = prob.reshape(_L * _K)
    Xs, Ps = _dispatch(x2, dest4, prob4)
    sh = _shared(x2, Ws1, bs1r, Ws2, bs2r)
    Y = _grouped(te.reshape(_NT), nact.reshape(1), Xs,
                 Ps.reshape(_NP, 1), W1, b1r, W2, b2r)
    out = _combine(Y, dest4, sh)
    return out.reshape(Bb, _L, _D)
